# baseline (speedup 1.0000x reference)
import jax
import jax.numpy as jnp
from jax.experimental import pallas as pl

_BLOCK = 2000


def _body(xb_ref, xm_ref, w1t_ref, b1_ref, w2t_ref, b2_ref, ob_ref, om_ref):
    ob_ref[...] = (
        jnp.dot(xb_ref[...], w1t_ref[...], preferred_element_type=jnp.float32)
        + b1_ref[...]
    )
    om_ref[...] = (
        jnp.dot(xm_ref[...], w2t_ref[...], preferred_element_type=jnp.float32)
        + b2_ref[...]
    )


def kernel(book_user_embeddings, movie_user_embeddings, W1, b1, W2, b2):
    n, d = book_user_embeddings.shape
    grid = (n // _BLOCK,)
    row_spec = pl.BlockSpec((_BLOCK, d), lambda i: (i, 0))
    full_spec = pl.BlockSpec((d, d), lambda i: (0, 0))
    bias_spec = pl.BlockSpec((1, d), lambda i: (0, 0))
    out_shape = jax.ShapeDtypeStruct((n, d), jnp.float32)
    book_out, movie_out = pl.pallas_call(
        _body,
        grid=grid,
        in_specs=[row_spec, row_spec, full_spec, bias_spec, full_spec, bias_spec],
        out_specs=[row_spec, row_spec],
        out_shape=[out_shape, out_shape],
    )(
        book_user_embeddings,
        movie_user_embeddings,
        W1.T,
        b1.reshape(1, d),
        W2.T,
        b2.reshape(1, d),
    )
    return (book_out, movie_out)


# speedup vs baseline: 1.1281x; 1.1281x over previous
import jax
import jax.numpy as jnp
from jax.experimental import pallas as pl

_BLOCK = 5000


def _body(xb_ref, xm_ref, w1t_ref, b1_ref, w2t_ref, b2_ref, ob_ref, om_ref):
    ob_ref[...] = (
        jnp.dot(xb_ref[...], w1t_ref[...], preferred_element_type=jnp.float32)
        + b1_ref[...]
    )
    om_ref[...] = (
        jnp.dot(xm_ref[...], w2t_ref[...], preferred_element_type=jnp.float32)
        + b2_ref[...]
    )


def kernel(book_user_embeddings, movie_user_embeddings, W1, b1, W2, b2):
    n, d = book_user_embeddings.shape
    grid = (n // _BLOCK,)
    row_spec = pl.BlockSpec((_BLOCK, d), lambda i: (i, 0))
    full_spec = pl.BlockSpec((d, d), lambda i: (0, 0))
    bias_spec = pl.BlockSpec((1, d), lambda i: (0, 0))
    out_shape = jax.ShapeDtypeStruct((n, d), jnp.float32)
    book_out, movie_out = pl.pallas_call(
        _body,
        grid=grid,
        in_specs=[row_spec, row_spec, full_spec, bias_spec, full_spec, bias_spec],
        out_specs=[row_spec, row_spec],
        out_shape=[out_shape, out_shape],
    )(
        book_user_embeddings,
        movie_user_embeddings,
        W1.T,
        b1.reshape(1, d),
        W2.T,
        b2.reshape(1, d),
    )
    return (book_out, movie_out)


# speedup vs baseline: 1.1840x; 1.0495x over previous
import jax
import jax.numpy as jnp
from jax.experimental import pallas as pl

_BLOCK = 10000


def _body(xb_ref, xm_ref, w1t_ref, b1_ref, w2t_ref, b2_ref, ob_ref, om_ref):
    ob_ref[...] = (
        jnp.dot(xb_ref[...], w1t_ref[...], preferred_element_type=jnp.float32)
        + b1_ref[...]
    )
    om_ref[...] = (
        jnp.dot(xm_ref[...], w2t_ref[...], preferred_element_type=jnp.float32)
        + b2_ref[...]
    )


def kernel(book_user_embeddings, movie_user_embeddings, W1, b1, W2, b2):
    n, d = book_user_embeddings.shape
    grid = (n // _BLOCK,)
    row_spec = pl.BlockSpec((_BLOCK, d), lambda i: (i, 0))
    full_spec = pl.BlockSpec((d, d), lambda i: (0, 0))
    bias_spec = pl.BlockSpec((1, d), lambda i: (0, 0))
    out_shape = jax.ShapeDtypeStruct((n, d), jnp.float32)
    book_out, movie_out = pl.pallas_call(
        _body,
        grid=grid,
        in_specs=[row_spec, row_spec, full_spec, bias_spec, full_spec, bias_spec],
        out_specs=[row_spec, row_spec],
        out_shape=[out_shape, out_shape],
    )(
        book_user_embeddings,
        movie_user_embeddings,
        W1.T,
        b1.reshape(1, d),
        W2.T,
        b2.reshape(1, d),
    )
    return (book_out, movie_out)


# speedup vs baseline: 1.2133x; 1.0247x over previous
import jax
import jax.numpy as jnp
from jax.experimental import pallas as pl

_BLOCK = 14000


def _body(xb_ref, xm_ref, w1t_ref, b1_ref, w2t_ref, b2_ref, ob_ref, om_ref):
    ob_ref[...] = (
        jnp.dot(xb_ref[...], w1t_ref[...], preferred_element_type=jnp.float32)
        + b1_ref[...]
    )
    om_ref[...] = (
        jnp.dot(xm_ref[...], w2t_ref[...], preferred_element_type=jnp.float32)
        + b2_ref[...]
    )


def kernel(book_user_embeddings, movie_user_embeddings, W1, b1, W2, b2):
    n, d = book_user_embeddings.shape
    grid = (pl.cdiv(n, _BLOCK),)
    row_spec = pl.BlockSpec((_BLOCK, d), lambda i: (i, 0))
    full_spec = pl.BlockSpec((d, d), lambda i: (0, 0))
    bias_spec = pl.BlockSpec((1, d), lambda i: (0, 0))
    out_shape = jax.ShapeDtypeStruct((n, d), jnp.float32)
    book_out, movie_out = pl.pallas_call(
        _body,
        grid=grid,
        in_specs=[row_spec, row_spec, full_spec, bias_spec, full_spec, bias_spec],
        out_specs=[row_spec, row_spec],
        out_shape=[out_shape, out_shape],
    )(
        book_user_embeddings,
        movie_user_embeddings,
        W1.T,
        b1.reshape(1, d),
        W2.T,
        b2.reshape(1, d),
    )
    return (book_out, movie_out)


# speedup vs baseline: 1.2250x; 1.0096x over previous
import jax
import jax.numpy as jnp
from jax.experimental import pallas as pl
from jax.experimental.pallas import tpu as pltpu

_BLOCK = 15200


def _body(xb_ref, xm_ref, w1t_ref, b1_ref, w2t_ref, b2_ref, ob_ref, om_ref):
    ob_ref[...] = (
        jnp.dot(xb_ref[...], w1t_ref[...], preferred_element_type=jnp.float32)
        + b1_ref[...]
    )
    om_ref[...] = (
        jnp.dot(xm_ref[...], w2t_ref[...], preferred_element_type=jnp.float32)
        + b2_ref[...]
    )


def kernel(book_user_embeddings, movie_user_embeddings, W1, b1, W2, b2):
    n, d = book_user_embeddings.shape
    grid = (pl.cdiv(n, _BLOCK),)
    row_spec = pl.BlockSpec((_BLOCK, d), lambda i: (i, 0))
    full_spec = pl.BlockSpec((d, d), lambda i: (0, 0))
    bias_spec = pl.BlockSpec((1, d), lambda i: (0, 0))
    out_shape = jax.ShapeDtypeStruct((n, d), jnp.float32)
    book_out, movie_out = pl.pallas_call(
        _body,
        grid=grid,
        in_specs=[row_spec, row_spec, full_spec, bias_spec, full_spec, bias_spec],
        out_specs=[row_spec, row_spec],
        out_shape=[out_shape, out_shape],
        compiler_params=pltpu.CompilerParams(vmem_limit_bytes=64 * 1024 * 1024),
    )(
        book_user_embeddings,
        movie_user_embeddings,
        W1.T,
        b1.reshape(1, d),
        W2.T,
        b2.reshape(1, d),
    )
    return (book_out, movie_out)
